# baseline (speedup 1.0000x reference)
import jax
import jax.numpy as jnp
from jax import lax
from jax.experimental import pallas as pl
from jax.experimental.pallas import tpu as pltpu
from jax.experimental.pallas import tpu_sc as plsc

_LAMB = 0.7
_N = 10000
_D = 128
_LANES = 16

_NC = 2
_NS = 16
_NW = _NC * _NS
_PER_W = _N // _NW
_EXTRA = (_N - _NW * _PER_W) // 8
_MAXROWS = _PER_W + 8
_CHUNKS = (88, 88, 80, 56)
_OFFS = tuple(sum(_CHUNKS[:i]) for i in range(len(_CHUNKS)))
_NFULL = len(_CHUNKS)
_DEPTH = 3


def _mix_rows(a_v, b_v, lo, hi):
    @plsc.parallel_loop(lo, hi, unroll=4)
    def _(i):
        for j in range(_D // _LANES):
            sl = pl.ds(j * _LANES, _LANES)
            a_v[i, sl] = a_v[i, sl] * _LAMB + b_v[i, sl] * (1.0 - _LAMB)


def _mix_body(x_hbm, pair_hbm, out_hbm, idx_v, a_v, b_v, sems, esem, lsem,
              osem):
    wid = lax.axis_index("s") * _NC + lax.axis_index("c")
    has_extra = wid < _EXTRA
    base = wid * _PER_W + 8 * jnp.minimum(wid, _EXTRA)

    pltpu.sync_copy(pair_hbm.at[pl.ds(base, _PER_W)],
                    idx_v.at[pl.ds(0, _PER_W)])

    @pl.when(has_extra)
    def _():
        pltpu.sync_copy(pair_hbm.at[pl.ds(base + _PER_W, 8)],
                        idx_v.at[pl.ds(_PER_W, 8)])

    def fire(k):
        sl = pl.ds(_OFFS[k], _CHUNKS[k])
        return (
            pltpu.async_copy(x_hbm.at[idx_v.at[sl]], b_v.at[sl], sems[k]),
            pltpu.async_copy(x_hbm.at[pl.ds(base + _OFFS[k], _CHUNKS[k])],
                             a_v.at[sl], sems[k]),
        )

    def extra_descs(make):
        sl = pl.ds(_PER_W, 8)
        return (
            make(x_hbm.at[idx_v.at[sl]], b_v.at[sl], esem),
            make(x_hbm.at[pl.ds(base + _PER_W, 8)], a_v.at[sl], esem),
        )

    waves = [fire(k) for k in range(_DEPTH)]

    @pl.when(has_extra)
    def _():
        extra_descs(pltpu.async_copy)

    stores = []
    for k in range(_NFULL):
        for c in waves[k]:
            c.wait()
        if k + _DEPTH < _NFULL:
            waves.append(fire(k + _DEPTH))
        _mix_rows(a_v, b_v, _OFFS[k], _OFFS[k] + _CHUNKS[k])
        stores.append(pltpu.async_copy(
            a_v.at[pl.ds(_OFFS[k], _CHUNKS[k])],
            out_hbm.at[pl.ds(base + _OFFS[k], _CHUNKS[k])], osem))

    @pl.when(has_extra)
    def _():
        for c in extra_descs(pltpu.make_async_copy):
            c.wait()
        _mix_rows(a_v, b_v, _PER_W, _MAXROWS)
        pltpu.sync_copy(a_v.at[pl.ds(_PER_W, 8)],
                        out_hbm.at[pl.ds(base + _PER_W, 8)])

    for s in stores:
        s.wait()


@jax.jit
def _node_mixup_sc(x, pair_idx):
    mesh = plsc.VectorSubcoreMesh(core_axis_name="c", subcore_axis_name="s")
    call = pl.kernel(
        _mix_body,
        out_type=jax.ShapeDtypeStruct((_N, _D), jnp.float32),
        mesh=mesh,
        scratch_types=[
            pltpu.VMEM((_MAXROWS,), jnp.int32),
            pltpu.VMEM((_MAXROWS, _D), jnp.float32),
            pltpu.VMEM((_MAXROWS, _D), jnp.float32),
            [pltpu.SemaphoreType.DMA] * _NFULL,
            pltpu.SemaphoreType.DMA,
            pltpu.SemaphoreType.DMA,
            pltpu.SemaphoreType.DMA,
        ],
    )
    return call(x, pair_idx)


def kernel(x, y, edge_index, pair_idx):
    x_mix = _node_mixup_sc(x, pair_idx)
    new_y = y.astype(jnp.int32)
    return (x_mix, new_y, edge_index)


# speedup vs baseline: 1.0068x; 1.0068x over previous
import jax
import jax.numpy as jnp
from jax import lax
from jax.experimental import pallas as pl
from jax.experimental.pallas import tpu as pltpu
from jax.experimental.pallas import tpu_sc as plsc

_LAMB = 0.7
_N = 10000
_D = 128
_LANES = 16

_NC = 2
_NS = 16
_NW = _NC * _NS
_PER_W = _N // _NW
_EXTRA = (_N - _NW * _PER_W) // 8
_MAXROWS = _PER_W + 8
_CHUNKS = (104, 96, 88, 24)
_OFFS = tuple(sum(_CHUNKS[:i]) for i in range(len(_CHUNKS)))
_NFULL = len(_CHUNKS)
_DEPTH = 2


def _mix_rows(a_v, b_v, lo, hi):
    @plsc.parallel_loop(lo, hi, unroll=4)
    def _(i):
        for j in range(_D // _LANES):
            sl = pl.ds(j * _LANES, _LANES)
            a_v[i, sl] = a_v[i, sl] * _LAMB + b_v[i, sl] * (1.0 - _LAMB)


def _mix_body(x_hbm, pair_hbm, out_hbm, idx_v, a_v, b_v, sems, esem, lsem,
              osem):
    wid = lax.axis_index("s") * _NC + lax.axis_index("c")
    has_extra = wid < _EXTRA
    base = wid * _PER_W + 8 * jnp.minimum(wid, _EXTRA)

    pltpu.sync_copy(pair_hbm.at[pl.ds(base, _PER_W)],
                    idx_v.at[pl.ds(0, _PER_W)])

    @pl.when(has_extra)
    def _():
        pltpu.sync_copy(pair_hbm.at[pl.ds(base + _PER_W, 8)],
                        idx_v.at[pl.ds(_PER_W, 8)])

    def fire(k):
        sl = pl.ds(_OFFS[k], _CHUNKS[k])
        return (
            pltpu.async_copy(x_hbm.at[idx_v.at[sl]], b_v.at[sl], sems[k]),
            pltpu.async_copy(x_hbm.at[pl.ds(base + _OFFS[k], _CHUNKS[k])],
                             a_v.at[sl], sems[k]),
        )

    def extra_descs(make):
        sl = pl.ds(_PER_W, 8)
        return (
            make(x_hbm.at[idx_v.at[sl]], b_v.at[sl], esem),
            make(x_hbm.at[pl.ds(base + _PER_W, 8)], a_v.at[sl], esem),
        )

    waves = [fire(k) for k in range(_DEPTH)]

    @pl.when(has_extra)
    def _():
        extra_descs(pltpu.async_copy)

    stores = []
    for k in range(_NFULL):
        for c in waves[k]:
            c.wait()
        if k + _DEPTH < _NFULL:
            waves.append(fire(k + _DEPTH))
        _mix_rows(a_v, b_v, _OFFS[k], _OFFS[k] + _CHUNKS[k])
        stores.append(pltpu.async_copy(
            a_v.at[pl.ds(_OFFS[k], _CHUNKS[k])],
            out_hbm.at[pl.ds(base + _OFFS[k], _CHUNKS[k])], osem))

    @pl.when(has_extra)
    def _():
        for c in extra_descs(pltpu.make_async_copy):
            c.wait()
        _mix_rows(a_v, b_v, _PER_W, _MAXROWS)
        pltpu.sync_copy(a_v.at[pl.ds(_PER_W, 8)],
                        out_hbm.at[pl.ds(base + _PER_W, 8)])

    for s in stores:
        s.wait()


@jax.jit
def _node_mixup_sc(x, pair_idx):
    mesh = plsc.VectorSubcoreMesh(core_axis_name="c", subcore_axis_name="s")
    call = pl.kernel(
        _mix_body,
        out_type=jax.ShapeDtypeStruct((_N, _D), jnp.float32),
        mesh=mesh,
        scratch_types=[
            pltpu.VMEM((_MAXROWS,), jnp.int32),
            pltpu.VMEM((_MAXROWS, _D), jnp.float32),
            pltpu.VMEM((_MAXROWS, _D), jnp.float32),
            [pltpu.SemaphoreType.DMA] * _NFULL,
            pltpu.SemaphoreType.DMA,
            pltpu.SemaphoreType.DMA,
            pltpu.SemaphoreType.DMA,
        ],
    )
    return call(x, pair_idx)


def kernel(x, y, edge_index, pair_idx):
    x_mix = _node_mixup_sc(x, pair_idx)
    new_y = y.astype(jnp.int32)
    return (x_mix, new_y, edge_index)


# speedup vs baseline: 1.0206x; 1.0137x over previous
import jax
import jax.numpy as jnp
from jax import lax
from jax.experimental import pallas as pl
from jax.experimental.pallas import tpu as pltpu
from jax.experimental.pallas import tpu_sc as plsc

_LAMB = 0.7
_N = 10000
_D = 128
_LANES = 16

_NC = 2
_NS = 16
_NW = _NC * _NS
_PER_W = _N // _NW
_EXTRA = (_N - _NW * _PER_W) // 8
_MAXROWS = _PER_W + 8
_CHUNKS = (24, 112, 112, 64)
_OFFS = tuple(sum(_CHUNKS[:i]) for i in range(len(_CHUNKS)))
_NFULL = len(_CHUNKS)
_DEPTH = 2


def _mix_rows(a_v, b_v, lo, hi):
    @plsc.parallel_loop(lo, hi, unroll=4)
    def _(i):
        for j in range(_D // _LANES):
            sl = pl.ds(j * _LANES, _LANES)
            a_v[i, sl] = a_v[i, sl] * _LAMB + b_v[i, sl] * (1.0 - _LAMB)


def _mix_body(x_hbm, pair_hbm, out_hbm, idx_v, a_v, b_v, sems, esem, lsem,
              osem):
    wid = lax.axis_index("s") * _NC + lax.axis_index("c")
    has_extra = wid < _EXTRA
    base = wid * _PER_W + 8 * jnp.minimum(wid, _EXTRA)

    pltpu.sync_copy(pair_hbm.at[pl.ds(base, _PER_W)],
                    idx_v.at[pl.ds(0, _PER_W)])

    @pl.when(has_extra)
    def _():
        pltpu.sync_copy(pair_hbm.at[pl.ds(base + _PER_W, 8)],
                        idx_v.at[pl.ds(_PER_W, 8)])

    def fire(k):
        sl = pl.ds(_OFFS[k], _CHUNKS[k])
        return (
            pltpu.async_copy(x_hbm.at[idx_v.at[sl]], b_v.at[sl], sems[k]),
            pltpu.async_copy(x_hbm.at[pl.ds(base + _OFFS[k], _CHUNKS[k])],
                             a_v.at[sl], sems[k]),
        )

    def extra_descs(make):
        sl = pl.ds(_PER_W, 8)
        return (
            make(x_hbm.at[idx_v.at[sl]], b_v.at[sl], esem),
            make(x_hbm.at[pl.ds(base + _PER_W, 8)], a_v.at[sl], esem),
        )

    waves = [fire(k) for k in range(_DEPTH)]

    @pl.when(has_extra)
    def _():
        extra_descs(pltpu.async_copy)

    stores = []
    for k in range(_NFULL):
        for c in waves[k]:
            c.wait()
        if k + _DEPTH < _NFULL:
            waves.append(fire(k + _DEPTH))
        _mix_rows(a_v, b_v, _OFFS[k], _OFFS[k] + _CHUNKS[k])
        stores.append(pltpu.async_copy(
            a_v.at[pl.ds(_OFFS[k], _CHUNKS[k])],
            out_hbm.at[pl.ds(base + _OFFS[k], _CHUNKS[k])], osem))

    @pl.when(has_extra)
    def _():
        for c in extra_descs(pltpu.make_async_copy):
            c.wait()
        _mix_rows(a_v, b_v, _PER_W, _MAXROWS)
        pltpu.sync_copy(a_v.at[pl.ds(_PER_W, 8)],
                        out_hbm.at[pl.ds(base + _PER_W, 8)])

    for s in stores:
        s.wait()


@jax.jit
def _node_mixup_sc(x, pair_idx):
    mesh = plsc.VectorSubcoreMesh(core_axis_name="c", subcore_axis_name="s")
    call = pl.kernel(
        _mix_body,
        out_type=jax.ShapeDtypeStruct((_N, _D), jnp.float32),
        mesh=mesh,
        scratch_types=[
            pltpu.VMEM((_MAXROWS,), jnp.int32),
            pltpu.VMEM((_MAXROWS, _D), jnp.float32),
            pltpu.VMEM((_MAXROWS, _D), jnp.float32),
            [pltpu.SemaphoreType.DMA] * _NFULL,
            pltpu.SemaphoreType.DMA,
            pltpu.SemaphoreType.DMA,
            pltpu.SemaphoreType.DMA,
        ],
    )
    return call(x, pair_idx)


def kernel(x, y, edge_index, pair_idx):
    x_mix = _node_mixup_sc(x, pair_idx)
    new_y = y.astype(jnp.int32)
    return (x_mix, new_y, edge_index)


# speedup vs baseline: 1.0246x; 1.0040x over previous
import jax
import jax.numpy as jnp
from jax import lax
from jax.experimental import pallas as pl
from jax.experimental.pallas import tpu as pltpu
from jax.experimental.pallas import tpu_sc as plsc

_LAMB = 0.7
_N = 10000
_D = 128
_LANES = 16

_NC = 2
_NS = 16
_NW = _NC * _NS
_PER_W = _N // _NW
_EXTRA = (_N - _NW * _PER_W) // 8
_MAXROWS = _PER_W + 8
_CHUNKS = (24, 128, 128, 32)
_OFFS = tuple(sum(_CHUNKS[:i]) for i in range(len(_CHUNKS)))
_NFULL = len(_CHUNKS)
_DEPTH = 2


def _mix_rows(a_v, b_v, lo, hi):
    @plsc.parallel_loop(lo, hi, unroll=4)
    def _(i):
        for j in range(_D // _LANES):
            sl = pl.ds(j * _LANES, _LANES)
            a_v[i, sl] = a_v[i, sl] * _LAMB + b_v[i, sl] * (1.0 - _LAMB)


def _mix_body(x_hbm, pair_hbm, out_hbm, idx_v, a_v, b_v, sems, esem, lsem,
              osem):
    wid = lax.axis_index("s") * _NC + lax.axis_index("c")
    has_extra = wid < _EXTRA
    base = wid * _PER_W + 8 * jnp.minimum(wid, _EXTRA)

    pltpu.sync_copy(pair_hbm.at[pl.ds(base, _PER_W)],
                    idx_v.at[pl.ds(0, _PER_W)])

    @pl.when(has_extra)
    def _():
        pltpu.sync_copy(pair_hbm.at[pl.ds(base + _PER_W, 8)],
                        idx_v.at[pl.ds(_PER_W, 8)])

    def fire(k):
        sl = pl.ds(_OFFS[k], _CHUNKS[k])
        return (
            pltpu.async_copy(x_hbm.at[idx_v.at[sl]], b_v.at[sl], sems[k]),
            pltpu.async_copy(x_hbm.at[pl.ds(base + _OFFS[k], _CHUNKS[k])],
                             a_v.at[sl], sems[k]),
        )

    def extra_descs(make):
        sl = pl.ds(_PER_W, 8)
        return (
            make(x_hbm.at[idx_v.at[sl]], b_v.at[sl], esem),
            make(x_hbm.at[pl.ds(base + _PER_W, 8)], a_v.at[sl], esem),
        )

    waves = [fire(k) for k in range(_DEPTH)]

    @pl.when(has_extra)
    def _():
        extra_descs(pltpu.async_copy)

    stores = []
    for k in range(_NFULL):
        for c in waves[k]:
            c.wait()
        if k + _DEPTH < _NFULL:
            waves.append(fire(k + _DEPTH))
        _mix_rows(a_v, b_v, _OFFS[k], _OFFS[k] + _CHUNKS[k])
        stores.append(pltpu.async_copy(
            a_v.at[pl.ds(_OFFS[k], _CHUNKS[k])],
            out_hbm.at[pl.ds(base + _OFFS[k], _CHUNKS[k])], osem))

    @pl.when(has_extra)
    def _():
        for c in extra_descs(pltpu.make_async_copy):
            c.wait()
        _mix_rows(a_v, b_v, _PER_W, _MAXROWS)
        pltpu.sync_copy(a_v.at[pl.ds(_PER_W, 8)],
                        out_hbm.at[pl.ds(base + _PER_W, 8)])

    for s in stores:
        s.wait()


@jax.jit
def _node_mixup_sc(x, pair_idx):
    mesh = plsc.VectorSubcoreMesh(core_axis_name="c", subcore_axis_name="s")
    call = pl.kernel(
        _mix_body,
        out_type=jax.ShapeDtypeStruct((_N, _D), jnp.float32),
        mesh=mesh,
        scratch_types=[
            pltpu.VMEM((_MAXROWS,), jnp.int32),
            pltpu.VMEM((_MAXROWS, _D), jnp.float32),
            pltpu.VMEM((_MAXROWS, _D), jnp.float32),
            [pltpu.SemaphoreType.DMA] * _NFULL,
            pltpu.SemaphoreType.DMA,
            pltpu.SemaphoreType.DMA,
            pltpu.SemaphoreType.DMA,
        ],
    )
    return call(x, pair_idx)


def kernel(x, y, edge_index, pair_idx):
    x_mix = _node_mixup_sc(x, pair_idx)
    new_y = y.astype(jnp.int32)
    return (x_mix, new_y, edge_index)


# speedup vs baseline: 1.0268x; 1.0021x over previous
import jax
import jax.numpy as jnp
from jax import lax
from jax.experimental import pallas as pl
from jax.experimental.pallas import tpu as pltpu
from jax.experimental.pallas import tpu_sc as plsc

_LAMB = 0.7
_N = 10000
_D = 128
_LANES = 16

_NC = 2
_NS = 16
_NW = _NC * _NS
_PER_W = _N // _NW
_EXTRA = (_N - _NW * _PER_W) // 8
_MAXROWS = _PER_W + 8
_CHUNKS = (24, 128, 128, 32)
_OFFS = tuple(sum(_CHUNKS[:i]) for i in range(len(_CHUNKS)))
_NFULL = len(_CHUNKS)
_DEPTH = 2


def _mix_rows(a_v, b_v, lo, hi):
    @plsc.parallel_loop(lo, hi, unroll=4)
    def _(i):
        for j in range(_D // _LANES):
            sl = pl.ds(j * _LANES, _LANES)
            a_v[i, sl] = a_v[i, sl] * _LAMB + b_v[i, sl] * (1.0 - _LAMB)


def _mix_body(x_hbm, pair_hbm, out_hbm, idx_v, a_v, b_v, sems, esem, osem):
    wid = lax.axis_index("s") * _NC + lax.axis_index("c")
    has_extra = wid < _EXTRA
    base = wid * _PER_W + 8 * jnp.minimum(wid, _EXTRA)

    pltpu.sync_copy(pair_hbm.at[pl.ds(base, _PER_W)],
                    idx_v.at[pl.ds(0, _PER_W)])

    @pl.when(has_extra)
    def _():
        pltpu.sync_copy(pair_hbm.at[pl.ds(base + _PER_W, 8)],
                        idx_v.at[pl.ds(_PER_W, 8)])

    def fire(k):
        sl = pl.ds(_OFFS[k], _CHUNKS[k])
        return (
            pltpu.async_copy(x_hbm.at[idx_v.at[sl]], b_v.at[sl], sems[k]),
            pltpu.async_copy(x_hbm.at[pl.ds(base + _OFFS[k], _CHUNKS[k])],
                             a_v.at[sl], sems[k]),
        )

    def extra_descs(make):
        sl = pl.ds(_PER_W, 8)
        return (
            make(x_hbm.at[idx_v.at[sl]], b_v.at[sl], esem),
            make(x_hbm.at[pl.ds(base + _PER_W, 8)], a_v.at[sl], esem),
        )

    waves = [fire(k) for k in range(_DEPTH)]

    @pl.when(has_extra)
    def _():
        extra_descs(pltpu.async_copy)

    stores = []
    for k in range(_NFULL):
        for c in waves[k]:
            c.wait()
        if k + _DEPTH < _NFULL:
            waves.append(fire(k + _DEPTH))
        _mix_rows(a_v, b_v, _OFFS[k], _OFFS[k] + _CHUNKS[k])
        stores.append(pltpu.async_copy(
            a_v.at[pl.ds(_OFFS[k], _CHUNKS[k])],
            out_hbm.at[pl.ds(base + _OFFS[k], _CHUNKS[k])], osem))

    @pl.when(has_extra)
    def _():
        for c in extra_descs(pltpu.make_async_copy):
            c.wait()
        _mix_rows(a_v, b_v, _PER_W, _MAXROWS)
        pltpu.sync_copy(a_v.at[pl.ds(_PER_W, 8)],
                        out_hbm.at[pl.ds(base + _PER_W, 8)])

    for s in stores:
        s.wait()


@jax.jit
def _node_mixup_sc(x, pair_idx):
    mesh = plsc.VectorSubcoreMesh(core_axis_name="c", subcore_axis_name="s")
    call = pl.kernel(
        _mix_body,
        out_type=jax.ShapeDtypeStruct((_N, _D), jnp.float32),
        mesh=mesh,
        scratch_types=[
            pltpu.VMEM((_MAXROWS,), jnp.int32),
            pltpu.VMEM((_MAXROWS, _D), jnp.float32),
            pltpu.VMEM((_MAXROWS, _D), jnp.float32),
            [pltpu.SemaphoreType.DMA] * _NFULL,
            pltpu.SemaphoreType.DMA,
            pltpu.SemaphoreType.DMA,
        ],
    )
    return call(x, pair_idx)


def kernel(x, y, edge_index, pair_idx):
    x_mix = _node_mixup_sc(x, pair_idx)
    new_y = y.astype(jnp.int32)
    return (x_mix, new_y, edge_index)
